# baseline (speedup 1.0000x reference)
import functools

import jax
import jax.numpy as jnp
from jax.experimental import pallas as pl
from jax.experimental.pallas import tpu as pltpu


def _gather_mask_body(ti_ref, tj_ref, a01i, a01j, a1i, a1j, a012i, a012j,
                      xi_ref, xj_ref, m0_ref, m1_ref, m2_ref, xij_ref):
    c01 = a01i[0] & a01j[0]
    c1 = a1i[0] & a1j[0]
    c012 = a012i[0] & a012j[0]
    m0_ref[0] = (c01 & jnp.logical_not(c1)).astype(jnp.int8)
    m1_ref[0] = c1.astype(jnp.int8)
    m2_ref[0] = (c012 & jnp.logical_not(c01)).astype(jnp.int8)
    xij_ref[0] = xi_ref[0] * xj_ref[0]


def _spmm_body(m0_ref, m1_ref, m2_ref, xij_ref, x_ref, wt_ref, b_ref, out_ref):
    d = x_ref.shape[1]
    acc = jnp.dot(xij_ref[0], wt_ref[0:d, :], preferred_element_type=jnp.float32)
    for k, mref in enumerate((m0_ref, m1_ref, m2_ref)):
        mk = mref[0].astype(jnp.float32)
        t = jnp.dot(mk, x_ref[...], preferred_element_type=jnp.float32)
        acc = acc + jnp.dot(t, wt_ref[(k + 1) * d:(k + 2) * d, :],
                            preferred_element_type=jnp.float32)
    out_ref[0] = acc + b_ref[0]


@functools.partial(jax.jit, static_argnames=("interpret",))
def kernel(x, adj_0_1, adj_1, adj_0_1_2, tar_ei, W, b, interpret=False):
    n, d = x.shape
    bsz = tar_ei.shape[1]
    out_dim = W.shape[0]
    ti = tar_ei[0].astype(jnp.int32)
    tj = tar_ei[1].astype(jnp.int32)

    a01 = adj_0_1.reshape(n, 1, n)
    a1 = adj_1.reshape(n, 1, n)
    a012 = adj_0_1_2.reshape(n, 1, n)
    x3 = x.reshape(n, 1, d)

    def row_spec(which):
        if which == 0:
            return pl.BlockSpec((1, 1, n), lambda i, ti, tj: (ti[i], 0, 0))
        return pl.BlockSpec((1, 1, n), lambda i, ti, tj: (tj[i], 0, 0))

    def xrow_spec(which):
        if which == 0:
            return pl.BlockSpec((1, 1, d), lambda i, ti, tj: (ti[i], 0, 0))
        return pl.BlockSpec((1, 1, d), lambda i, ti, tj: (tj[i], 0, 0))

    out_row = pl.BlockSpec((1, 1, n), lambda i, ti, tj: (i, 0, 0))
    out_xij = pl.BlockSpec((1, 1, d), lambda i, ti, tj: (i, 0, 0))

    m0, m1, m2, xij = pl.pallas_call(
        _gather_mask_body,
        grid_spec=pltpu.PrefetchScalarGridSpec(
            num_scalar_prefetch=2,
            grid=(bsz,),
            in_specs=[row_spec(0), row_spec(1), row_spec(0), row_spec(1),
                      row_spec(0), row_spec(1), xrow_spec(0), xrow_spec(1)],
            out_specs=[out_row, out_row, out_row, out_xij],
        ),
        out_shape=[
            jax.ShapeDtypeStruct((bsz, 1, n), jnp.int8),
            jax.ShapeDtypeStruct((bsz, 1, n), jnp.int8),
            jax.ShapeDtypeStruct((bsz, 1, n), jnp.int8),
            jax.ShapeDtypeStruct((bsz, 1, d), jnp.float32),
        ],
        interpret=interpret,
    )(ti, tj, a01, a01, a1, a1, a012, a012, x3, x3)

    bg = 128 if bsz % 128 == 0 else bsz
    nb = bsz // bg
    m0r = m0.reshape(nb, bg, n)
    m1r = m1.reshape(nb, bg, n)
    m2r = m2.reshape(nb, bg, n)
    xijr = xij.reshape(nb, bg, d)

    mask_spec = pl.BlockSpec((1, bg, n), lambda i: (i, 0, 0))
    xij_spec = pl.BlockSpec((1, bg, d), lambda i: (i, 0, 0))
    x_spec = pl.BlockSpec((n, d), lambda i: (0, 0))
    wt_spec = pl.BlockSpec((4 * d, out_dim), lambda i: (0, 0))
    b_spec = pl.BlockSpec((1, out_dim), lambda i: (0, 0))

    out = pl.pallas_call(
        _spmm_body,
        grid=(nb,),
        in_specs=[mask_spec, mask_spec, mask_spec, xij_spec, x_spec, wt_spec,
                  b_spec],
        out_specs=pl.BlockSpec((1, bg, out_dim), lambda i: (i, 0, 0)),
        out_shape=jax.ShapeDtypeStruct((nb, bg, out_dim), jnp.float32),
        interpret=interpret,
    )(m0r, m1r, m2r, xijr, x, W.T, b.reshape(1, out_dim))

    return out.reshape(bsz, out_dim)
